# baseline (speedup 1.0000x reference)
import functools

import jax
import jax.numpy as jnp
from jax import lax
from jax.experimental import pallas as pl
from jax.experimental.pallas import tpu as pltpu
from jax.experimental.pallas import tpu_sc as plsc

VOCAB = 1000
D = 1000
DP = 1024
NG = DP // 128
SEQP = 56
B = 1024 * SEQP
NC, NS = 2, 16
NW = NC * NS
BPW = B // NW
CH = 16
NCHUNK = BPW // CH
NB = 2

_mesh = plsc.VectorSubcoreMesh(core_axis_name="c", subcore_axis_name="s")


@functools.partial(
    pl.kernel,
    mesh=_mesh,
    out_type=jax.ShapeDtypeStruct((B, D), jnp.float32),
    compiler_params=pltpu.CompilerParams(use_tc_tiling_on_sc=True),
    scratch_types=[
        pltpu.VMEM((NCHUNK, CH), jnp.int32),
        pltpu.VMEM((CH, D), jnp.float32),
        pltpu.VMEM((CH, D), jnp.float32),
        pltpu.SemaphoreType.DMA,
        pltpu.SemaphoreType.DMA,
        pltpu.SemaphoreType.DMA,
        pltpu.SemaphoreType.DMA,
    ],
)
def _emb_gather(idx_hbm, table_hbm, out_hbm, idx_v, buf0, buf1,
                gsem0, gsem1, osem0, osem1):
    wid = lax.axis_index("s") * NC + lax.axis_index("c")
    base = wid * BPW
    bufs = (buf0, buf1)
    gsems = (gsem0, gsem1)
    osems = (osem0, osem1)

    pltpu.sync_copy(idx_hbm.at[wid], idx_v)

    def start_gathers(b, i):
        row8 = idx_v[i] * 8

        def gath(j, carry):
            pltpu.async_copy(table_hbm.at[row8 + j],
                             bufs[b].at[:, pl.ds(128 * j, 128)], gsems[b])
            return carry

        lax.fori_loop(0, NG, gath, 0)

    def wait_gathers(b, i):
        row8 = idx_v[i] * 8

        def gath(j, carry):
            pltpu.make_async_copy(table_hbm.at[row8 + j],
                                  bufs[b].at[:, pl.ds(128 * j, 128)],
                                  gsems[b]).wait()
            return carry

        lax.fori_loop(0, NG, gath, 0)

    for b in range(NB):
        start_gathers(b, b)

    def body(j, carry):
        for b in range(NB):
            i = NB * j + b
            dst = out_hbm.at[pl.ds(base + i * CH, CH)]
            wait_gathers(b, i)
            pltpu.async_copy(bufs[b], dst, osems[b])

            @pl.when(i + NB < NCHUNK)
            def _():
                pltpu.make_async_copy(bufs[b], dst, osems[b]).wait()
                start_gathers(b, i + NB)
        return carry

    lax.fori_loop(0, NCHUNK // NB, body, 0)

    for b in range(NB):
        i = NCHUNK - NB + b
        pltpu.make_async_copy(bufs[b], out_hbm.at[pl.ds(base + i * CH, CH)],
                              osems[b]).wait()


def kernel(x, emb):
    table_v = jnp.pad(emb, ((0, 0), (0, DP - D))).reshape(VOCAB * NG, 128)
    idx = jnp.pad(x.astype(jnp.int32), ((0, 0), (0, SEQP - x.shape[1])))
    idx = idx.reshape(NW, NCHUNK, CH)
    out = _emb_gather(idx, table_v)
    return out.reshape(x.shape[0], SEQP, D)[:, :x.shape[1], :]


# speedup vs baseline: 1.5369x; 1.5369x over previous
import functools

import jax
import jax.numpy as jnp
from jax import lax
from jax.experimental import pallas as pl
from jax.experimental.pallas import tpu as pltpu
from jax.experimental.pallas import tpu_sc as plsc

VOCAB = 1000
D = 1000
DP = 1024
NG = DP // 128
B = 1024 * 50
NC, NS = 2, 16
NW = NC * NS
BPW = B // NW
CH = 16
NCHUNK = BPW // CH
NB = 2

_mesh = plsc.VectorSubcoreMesh(core_axis_name="c", subcore_axis_name="s")


@functools.partial(
    pl.kernel,
    mesh=_mesh,
    out_type=jax.ShapeDtypeStruct((B, D), jnp.float32),
    compiler_params=pltpu.CompilerParams(use_tc_tiling_on_sc=True),
    scratch_types=[
        pltpu.VMEM((NCHUNK, CH), jnp.int32),
        pltpu.VMEM((CH, D), jnp.float32),
        pltpu.VMEM((CH, D), jnp.float32),
        pltpu.SemaphoreType.DMA,
        pltpu.SemaphoreType.DMA,
        pltpu.SemaphoreType.DMA,
        pltpu.SemaphoreType.DMA,
    ],
)
def _emb_gather(idx_hbm, table_hbm, out_hbm, idx_v, buf0, buf1,
                gsem0, gsem1, osem0, osem1):
    wid = lax.axis_index("s") * NC + lax.axis_index("c")
    base = wid * BPW
    bufs = (buf0, buf1)
    gsems = (gsem0, gsem1)
    osems = (osem0, osem1)

    pltpu.sync_copy(idx_hbm.at[wid], idx_v)

    def start_gathers(b, i):
        row8 = idx_v[i] * 8

        def gath(j, carry):
            pltpu.async_copy(table_hbm.at[row8 + j],
                             bufs[b].at[:, pl.ds(128 * j, 128)], gsems[b])
            return carry

        lax.fori_loop(0, NG, gath, 0)

    def wait_gathers(b, i):
        row8 = idx_v[i] * 8

        def gath(j, carry):
            pltpu.make_async_copy(table_hbm.at[row8 + j],
                                  bufs[b].at[:, pl.ds(128 * j, 128)],
                                  gsems[b]).wait()
            return carry

        lax.fori_loop(0, NG, gath, 0)

    for b in range(NB):
        start_gathers(b, b)

    def body(j, carry):
        for b in range(NB):
            i = NB * j + b
            dst = out_hbm.at[pl.ds(base + i * CH, CH)]
            wait_gathers(b, i)
            pltpu.async_copy(bufs[b], dst, osems[b])

            @pl.when(i + NB < NCHUNK)
            def _():
                pltpu.make_async_copy(bufs[b], dst, osems[b]).wait()
                start_gathers(b, i + NB)
        return carry

    lax.fori_loop(0, NCHUNK // NB, body, 0)

    for b in range(NB):
        i = NCHUNK - NB + b
        pltpu.make_async_copy(bufs[b], out_hbm.at[pl.ds(base + i * CH, CH)],
                              osems[b]).wait()


def kernel(x, emb):
    table_v = jnp.pad(emb, ((0, 0), (0, DP - D))).reshape(VOCAB * NG, 128)
    idx = x.reshape(NW, NCHUNK, CH).astype(jnp.int32)
    out = _emb_gather(idx, table_v)
    return out.reshape(x.shape[0], x.shape[1], D)


# speedup vs baseline: 2.0550x; 1.3371x over previous
import functools

import jax
import jax.numpy as jnp
from jax import lax
from jax.experimental import pallas as pl
from jax.experimental.pallas import tpu as pltpu
from jax.experimental.pallas import tpu_sc as plsc

VOCAB = 1000
D = 1000
DP = 1024
NG = DP // 128
SEQP = 56
B = 1024 * SEQP
NC, NS = 2, 16
NW = NC * NS
BPW = B // NW
CH = 16
NCHUNK = BPW // CH
NB = 2

_mesh = plsc.VectorSubcoreMesh(core_axis_name="c", subcore_axis_name="s")


@functools.partial(
    pl.kernel,
    mesh=_mesh,
    out_type=jax.ShapeDtypeStruct((B, D), jnp.float32),
    compiler_params=pltpu.CompilerParams(use_tc_tiling_on_sc=True),
    scratch_types=[
        pltpu.VMEM((NCHUNK, CH), jnp.int32),
        pltpu.VMEM((CH, D), jnp.float32),
        pltpu.VMEM((CH, D), jnp.float32),
        pltpu.SemaphoreType.DMA,
        pltpu.SemaphoreType.DMA,
        pltpu.SemaphoreType.DMA,
        pltpu.SemaphoreType.DMA,
    ],
)
def _emb_gather(idx_hbm, table_hbm, out_hbm, idx_v, buf0, buf1,
                gsem0, gsem1, osem0, osem1):
    wid = lax.axis_index("s") * NC + lax.axis_index("c")
    base = wid * BPW
    bufs = (buf0, buf1)
    gsems = (gsem0, gsem1)
    osems = (osem0, osem1)

    pltpu.sync_copy(idx_hbm.at[wid], idx_v)

    def start_gathers(b, i):
        row8 = idx_v[i] * 8

        def gath(j, carry):
            pltpu.async_copy(table_hbm.at[row8 + j],
                             bufs[b].at[:, pl.ds(128 * j, 128)], gsems[b])
            return carry

        lax.fori_loop(0, NG, gath, 0)

    def wait_gathers(b, i):
        row8 = idx_v[i] * 8

        def gath(j, carry):
            pltpu.make_async_copy(table_hbm.at[row8 + j],
                                  bufs[b].at[:, pl.ds(128 * j, 128)],
                                  gsems[b]).wait()
            return carry

        lax.fori_loop(0, NG, gath, 0)

    for b in range(NB):
        start_gathers(b, b)

    def body(j, carry):
        for b in range(NB):
            i = NB * j + b
            dst = out_hbm.at[pl.ds(base + i * CH, CH)]
            wait_gathers(b, i)
            pltpu.async_copy(bufs[b], dst, osems[b])

            @pl.when(i + NB < NCHUNK)
            def _():
                pltpu.make_async_copy(bufs[b], dst, osems[b]).wait()
                start_gathers(b, i + NB)
        return carry

    lax.fori_loop(0, NCHUNK // NB, body, 0)

    for b in range(NB):
        i = NCHUNK - NB + b
        pltpu.make_async_copy(bufs[b], out_hbm.at[pl.ds(base + i * CH, CH)],
                              osems[b]).wait()


def kernel(x, emb):
    table_v = jnp.pad(emb, ((0, 0), (0, DP - D))).reshape(VOCAB * NG, 128)
    idx = jnp.pad(x.astype(jnp.int32), ((0, 0), (0, SEQP - x.shape[1])),
                  mode='edge')
    idx = idx.reshape(NW, NCHUNK, CH)
    out = _emb_gather(idx, table_v)
    return out.reshape(x.shape[0], SEQP, D)[:, :x.shape[1], :]
